# baseline (speedup 1.0000x reference)
import functools

import jax
import jax.numpy as jnp
from jax import lax
from jax.experimental import pallas as pl
from jax.experimental.pallas import tpu as pltpu
from jax.experimental.pallas import tpu_sc as plsc

N = 10000
D = 128
E = 320000
NC, NS, L = 2, 16, 16
NW = NC * NS
CHUNK = 128
NCHUNKS = E // CHUNK
WIDTH = D + L
NPAD = 10240
ROWS_PER_TILE = NPAD // NS
BLK = 512
GRID = (N + BLK - 1) // BLK

_HI = lax.Precision.DEFAULT


def _prep_body(fa, fb, wm0, wm1, wm2, wm3, bm0, bm1, bm2, bm3,
               a10, a11, a12, a13, a20, a21, a22, a23,
               t0, t1, t2, t3, vecs):
    feats_src = (fa, fa, fb, fb)
    feats_tbl = (fb, fb, fa, fa)
    wms = (wm0, wm1, wm2, wm3)
    bms = (bm0, bm1, bm2, bm3)
    a1s = (a10, a11, a12, a13)
    a2s = (a20, a21, a22, a23)
    touts = (t0, t1, t2, t3)
    rows = []
    for i in range(4):
        tbl = jnp.dot(feats_tbl[i][...], wms[i][...], precision=_HI,
                      preferred_element_type=jnp.float32) + bms[i][...]
        touts[i][...] = tbl
        sa = lax.dot_general(a1s[i][...], feats_src[i][...],
                             (((1,), (1,)), ((), ())), precision=_HI,
                             preferred_element_type=jnp.float32)
        sb = lax.dot_general(a2s[i][...], tbl,
                             (((1,), (1,)), ((), ())), precision=_HI,
                             preferred_element_type=jnp.float32)
        rows.append(sa)
        rows.append(sb)
    vecs[...] = jnp.concatenate(rows, axis=0)


def _prep(fa, fb, wms, bms, a1s, a2s):
    full = lambda shape: pl.BlockSpec(shape, lambda i: tuple(0 for _ in shape))
    rowblk = pl.BlockSpec((BLK, D), lambda i: (i, 0))
    out_shapes = ([jax.ShapeDtypeStruct((N, D), jnp.float32)] * 4
                  + [jax.ShapeDtypeStruct((8, N), jnp.float32)])
    out_specs = ([rowblk] * 4 + [pl.BlockSpec((8, BLK), lambda i: (0, i))])
    return pl.pallas_call(
        _prep_body,
        grid=(GRID,),
        in_specs=[rowblk, rowblk] + [full((D, D))] * 4 + [full((1, D))] * 4
                 + [full((1, D))] * 8,
        out_specs=out_specs,
        out_shape=out_shapes,
    )(fa, fb, *wms, *bms, *a1s, *a2s)


def _sc_body(edges, sa_h, sb_h, tbl_h, out_acc, out_rs,
             ib, sc_idx, sa_v, sb_v, w_v, rows_v, zero_v, zrs_v,
             sem_i, sem_g, sem_s, acc_sh, rs_sh):
    cid = lax.axis_index("c")
    sid = lax.axis_index("s")
    wid = sid * NC + cid

    for r in range(L):
        for g in range(D // L):
            zero_v[r, pl.ds(g * L, L)] = jnp.zeros((L,), jnp.float32)
    for g in range(ROWS_PER_TILE // L):
        zrs_v[pl.ds(g * L, L)] = jnp.zeros((L,), jnp.float32)

    def zero_body(t, _):
        pltpu.sync_copy(zero_v, acc_sh.at[pl.ds(sid * ROWS_PER_TILE + t * L, L)])
        return 0
    lax.fori_loop(0, ROWS_PER_TILE // L, zero_body, 0)
    pltpu.sync_copy(zrs_v, rs_sh.at[pl.ds(sid * ROWS_PER_TILE, ROWS_PER_TILE)])
    plsc.subcore_barrier()

    n_my = NCHUNKS // NW + jnp.where(wid < NCHUNKS % NW, 1, 0)

    _bcast_dnums = lax.GatherDimensionNumbers(
        offset_dims=(), collapsed_slice_dims=(0,), start_index_map=(0,))

    def issue_idx(c, b):
        off = (wid + c * NW) * CHUNK
        pltpu.async_copy(edges.at[0, pl.ds(off, CHUNK)], ib[b].at[0], sem_i[b])
        pltpu.async_copy(edges.at[1, pl.ds(off, CHUNK)], ib[b].at[1], sem_i[b])

    def wait_idx(b):
        pltpu.make_async_copy(edges.at[0, pl.ds(0, CHUNK)], ib[b].at[0],
                              sem_i[b]).wait()
        pltpu.make_async_copy(edges.at[1, pl.ds(0, CHUNK)], ib[b].at[1],
                              sem_i[b]).wait()

    def issue_gathers(b):
        pltpu.async_copy(sa_h.at[ib[b].at[0]], sa_v[b], sem_g[b])
        pltpu.async_copy(sb_h.at[ib[b].at[1]], sb_v[b], sem_g[b])
        pltpu.async_copy(tbl_h.at[ib[b].at[1]], rows_v[b], sem_g[b])

    def wait_gathers(b):
        pltpu.make_async_copy(sa_h.at[ib[b].at[0]], sa_v[b], sem_g[b]).wait()
        pltpu.make_async_copy(sb_h.at[ib[b].at[1]], sb_v[b], sem_g[b]).wait()
        pltpu.make_async_copy(tbl_h.at[ib[b].at[1]], rows_v[b], sem_g[b]).wait()

    def issue_scatter(b):
        pltpu.async_copy(rows_v[b], acc_sh.at[sc_idx[b]], sem_s[b], add=True)
        pltpu.async_copy(w_v[b], rs_sh.at[sc_idx[b]], sem_s[b], add=True)

    def wait_scatter(b):
        pltpu.make_async_copy(rows_v[b], acc_sh.at[sc_idx[b]], sem_s[b],
                              ).wait()
        pltpu.make_async_copy(w_v[b], rs_sh.at[sc_idx[b]], sem_s[b]).wait()

    issue_idx(0, 0)
    wait_idx(0)
    issue_gathers(0)

    @pl.when(1 < n_my)
    def _p1():
        issue_idx(1, 1)

    def pair_body(g, _):
        for b in (0, 1):
            c = 2 * g + b
            o = 1 - b

            @pl.when(c < n_my)
            def _process():
                wait_gathers(b)

                wregs = []
                for gi in range(CHUNK // L):
                    sc_idx[b][pl.ds(gi * L, L)] = ib[b][0, pl.ds(gi * L, L)]
                    z = sa_v[b][pl.ds(gi * L, L)] + sb_v[b][pl.ds(gi * L, L)]
                    el = jnp.where(z > 0, z, 0.1 * (jnp.exp(z) - 1.0))
                    w = jnp.exp(el)
                    w_v[b][pl.ds(gi * L, L)] = w
                    wregs.append(w)

                @pl.when(c + 2 < n_my)
                def _prefetch_idx():
                    issue_idx(c + 2, b)

                @pl.when(c >= 1)
                def _drain():
                    wait_scatter(o)

                @pl.when(c + 1 < n_my)
                def _prefetch():
                    wait_idx(o)
                    issue_gathers(o)

                for gi in range(CHUNK // L):
                    for u in range(L):
                        e = gi * L + u
                        wv = lax.gather(wregs[gi],
                                        jnp.full((L, 1), u, jnp.int32),
                                        _bcast_dnums, (1,),
                                        mode=lax.GatherScatterMode.PROMISE_IN_BOUNDS)
                        for kk in range(D // L):
                            rows_v[b][e, pl.ds(kk * L, L)] = (
                                rows_v[b][e, pl.ds(kk * L, L)] * wv)

                issue_scatter(b)
        return 0

    lax.fori_loop(0, (n_my + 1) // 2, pair_body, 0)

    @pl.when(n_my % 2 == 1)
    def _drain_even():
        wait_scatter(0)

    @pl.when(n_my % 2 == 0)
    def _drain_odd():
        wait_scatter(1)
    plsc.subcore_barrier()
    pltpu.sync_copy(acc_sh.at[pl.ds(sid * ROWS_PER_TILE, ROWS_PER_TILE)],
                    out_acc.at[cid, pl.ds(sid * ROWS_PER_TILE, ROWS_PER_TILE)])
    pltpu.sync_copy(rs_sh.at[pl.ds(sid * ROWS_PER_TILE, ROWS_PER_TILE)],
                    out_rs.at[cid, pl.ds(sid * ROWS_PER_TILE, ROWS_PER_TILE)])


_sc_agg = pl.kernel(
    _sc_body,
    out_type=[jax.ShapeDtypeStruct((NC, NPAD, D), jnp.float32),
              jax.ShapeDtypeStruct((NC, NPAD), jnp.float32)],
    mesh=plsc.VectorSubcoreMesh(core_axis_name="c", subcore_axis_name="s",
                                num_cores=NC, num_subcores=NS),
    compiler_params=pltpu.CompilerParams(needs_layout_passes=False),
    scratch_types=[
        [pltpu.VMEM((2, CHUNK), jnp.int32)] * 2,
        [pltpu.VMEM((CHUNK,), jnp.int32)] * 2,
        [pltpu.VMEM((CHUNK,), jnp.float32)] * 2,
        [pltpu.VMEM((CHUNK,), jnp.float32)] * 2,
        [pltpu.VMEM((CHUNK,), jnp.float32)] * 2,
        [pltpu.VMEM((CHUNK, D), jnp.float32)] * 2,
        pltpu.VMEM((L, D), jnp.float32),
        pltpu.VMEM((ROWS_PER_TILE,), jnp.float32),
        [pltpu.SemaphoreType.DMA] * 2,
        [pltpu.SemaphoreType.DMA] * 2,
        [pltpu.SemaphoreType.DMA] * 2,
        pltpu.VMEM_SHARED((NPAD, D), jnp.float32),
        pltpu.VMEM_SHARED((NPAD,), jnp.float32),
    ],
)


def _final_body(feat, accp, accn, rsp_r, rsn_r, w1a, w1b, w1c, bu1, pw, w2,
                bu2, out):
    rsp = rsp_r[0] + rsp_r[1]
    rsp = jnp.where(rsp == 0.0, 1.0, rsp)
    mp = (accp[0] + accp[1]) / rsp
    rsn = rsn_r[0] + rsn_r[1]
    rsn = jnp.where(rsn == 0.0, 1.0, rsn)
    mn = (accn[0] + accn[1]) / rsn
    h = (jnp.dot(feat[...], w1a[...], precision=_HI,
                 preferred_element_type=jnp.float32)
         + jnp.dot(mp, w1b[...], precision=_HI,
                   preferred_element_type=jnp.float32)
         + jnp.dot(mn, w1c[...], precision=_HI,
                   preferred_element_type=jnp.float32)
         + bu1[...])
    h = jnp.where(h > 0, h, pw[0, 0] * h)
    out[...] = jnp.dot(h, w2[...], precision=_HI,
                       preferred_element_type=jnp.float32) + bu2[...]


def _final(feat, accp, accn, rsp, rsn, w1a, w1b, w1c, bu1, pw, w2, bu2):
    full = lambda shape: pl.BlockSpec(shape, lambda i: tuple(0 for _ in shape))
    accblk = pl.BlockSpec((NC, BLK, D), lambda i: (0, i, 0))
    rsblk = pl.BlockSpec((NC, BLK, 1), lambda i: (0, i, 0))
    return pl.pallas_call(
        _final_body,
        grid=(GRID,),
        in_specs=[pl.BlockSpec((BLK, D), lambda i: (i, 0)), accblk, accblk,
                  rsblk, rsblk,
                  full((D, 2 * D)), full((D, 2 * D)), full((D, 2 * D)),
                  full((1, 2 * D)), full((1, 1)), full((2 * D, D)),
                  full((1, D))],
        out_specs=pl.BlockSpec((BLK, D), lambda i: (i, 0)),
        out_shape=jax.ShapeDtypeStruct((N, D), jnp.float32),
    )(feat, accp, accn, rsp, rsn, w1a, w1b, w1c, bu1, pw, w2, bu2)


def kernel(feature_a, feature_b, edge_ab_pos, edge_ab_neg, edge_ba_pos,
           edge_ba_neg, Wm_abp, bm_abp, a_abp, Wm_abn, bm_abn, a_abn,
           Wm_bap, bm_bap, a_bap, Wm_ban, bm_ban, a_ban,
           Wu1, bu1, pw, Wu2, bu2):
    f32 = jnp.float32
    wms = [Wm_abp, Wm_abn, Wm_bap, Wm_ban]
    bms = [b.reshape(1, D).astype(f32) for b in (bm_abp, bm_abn, bm_bap, bm_ban)]
    avs = [a_abp, a_abn, a_bap, a_ban]
    a1s = [a[:D, 0].reshape(1, D).astype(f32) for a in avs]
    a2s = [a[D:, 0].reshape(1, D).astype(f32) for a in avs]

    t0, t1, t2, t3, vecs = _prep(feature_a, feature_b, wms, bms, a1s, a2s)
    tables = (t0, t1, t2, t3)
    edges = (edge_ab_pos.astype(jnp.int32), edge_ab_neg.astype(jnp.int32),
             edge_ba_pos.astype(jnp.int32), edge_ba_neg.astype(jnp.int32))

    accs, rss = [], []
    for i in range(4):
        sa = vecs[2 * i]
        sb = vecs[2 * i + 1]
        acc, rs = _sc_agg(edges[i], sa, sb, tables[i])
        accs.append(acc)
        rss.append(rs.reshape(NC, NPAD, 1))

    w1a = Wu1[:D]
    w1b = Wu1[D:2 * D]
    w1c = Wu1[2 * D:]
    bu1r = bu1.reshape(1, 2 * D)
    bu2r = bu2.reshape(1, D)
    pwr = pw.reshape(1, 1)
    new_a = _final(feature_a, accs[0], accs[1], rss[0], rss[1], w1a, w1b,
                   w1c, bu1r, pwr, Wu2, bu2r)
    new_b = _final(feature_b, accs[2], accs[3], rss[2], rss[3], w1a, w1b,
                   w1c, bu1r, pwr, Wu2, bu2r)
    return (new_a, new_b)


# speedup vs baseline: 1.0041x; 1.0041x over previous
import functools

import jax
import jax.numpy as jnp
from jax import lax
from jax.experimental import pallas as pl
from jax.experimental.pallas import tpu as pltpu
from jax.experimental.pallas import tpu_sc as plsc

N = 10000
D = 128
E = 320000
NC, NS, L = 2, 16, 16
NW = NC * NS
CHUNK = 64
NCHUNKS = E // CHUNK
WIDTH = D + L
NPAD = 10240
ROWS_PER_TILE = NPAD // NS
BLK = 512
GRID = (N + BLK - 1) // BLK

_HI = lax.Precision.DEFAULT


def _prep_body(fa, fb, wm0, wm1, wm2, wm3, bm0, bm1, bm2, bm3,
               a10, a11, a12, a13, a20, a21, a22, a23,
               t0, t1, t2, t3, vecs):
    feats_src = (fa, fa, fb, fb)
    feats_tbl = (fb, fb, fa, fa)
    wms = (wm0, wm1, wm2, wm3)
    bms = (bm0, bm1, bm2, bm3)
    a1s = (a10, a11, a12, a13)
    a2s = (a20, a21, a22, a23)
    touts = (t0, t1, t2, t3)
    rows = []
    for i in range(4):
        tbl = jnp.dot(feats_tbl[i][...], wms[i][...], precision=_HI,
                      preferred_element_type=jnp.float32) + bms[i][...]
        touts[i][...] = tbl
        sa = lax.dot_general(a1s[i][...], feats_src[i][...],
                             (((1,), (1,)), ((), ())), precision=_HI,
                             preferred_element_type=jnp.float32)
        sb = lax.dot_general(a2s[i][...], tbl,
                             (((1,), (1,)), ((), ())), precision=_HI,
                             preferred_element_type=jnp.float32)
        rows.append(sa)
        rows.append(sb)
    vecs[...] = jnp.concatenate(rows, axis=0)


def _prep(fa, fb, wms, bms, a1s, a2s):
    full = lambda shape: pl.BlockSpec(shape, lambda i: tuple(0 for _ in shape))
    rowblk = pl.BlockSpec((BLK, D), lambda i: (i, 0))
    out_shapes = ([jax.ShapeDtypeStruct((N, D), jnp.float32)] * 4
                  + [jax.ShapeDtypeStruct((8, N), jnp.float32)])
    out_specs = ([rowblk] * 4 + [pl.BlockSpec((8, BLK), lambda i: (0, i))])
    return pl.pallas_call(
        _prep_body,
        grid=(GRID,),
        in_specs=[rowblk, rowblk] + [full((D, D))] * 4 + [full((1, D))] * 4
                 + [full((1, D))] * 8,
        out_specs=out_specs,
        out_shape=out_shapes,
    )(fa, fb, *wms, *bms, *a1s, *a2s)


def _sc_body(edges, sa_h, sb_h, tbl_h, out_acc, out_rs,
             ib, sc_idx, sa_v, sb_v, w_v, rows_v, scaled_v, zero_v, zrs_v,
             sem_i, sem_g, sem_s, acc_sh, rs_sh):
    cid = lax.axis_index("c")
    sid = lax.axis_index("s")
    wid = sid * NC + cid

    for r in range(L):
        for g in range(D // L):
            zero_v[r, pl.ds(g * L, L)] = jnp.zeros((L,), jnp.float32)
    for g in range(ROWS_PER_TILE // L):
        zrs_v[pl.ds(g * L, L)] = jnp.zeros((L,), jnp.float32)

    def zero_body(t, _):
        pltpu.sync_copy(zero_v, acc_sh.at[pl.ds(sid * ROWS_PER_TILE + t * L, L)])
        return 0
    lax.fori_loop(0, ROWS_PER_TILE // L, zero_body, 0)
    pltpu.sync_copy(zrs_v, rs_sh.at[pl.ds(sid * ROWS_PER_TILE, ROWS_PER_TILE)])
    plsc.subcore_barrier()

    n_my = NCHUNKS // NW + jnp.where(wid < NCHUNKS % NW, 1, 0)

    _bcast_dnums = lax.GatherDimensionNumbers(
        offset_dims=(), collapsed_slice_dims=(0,), start_index_map=(0,))

    def issue_idx(c, b):
        off = (wid + c * NW) * CHUNK
        pltpu.async_copy(edges.at[0, pl.ds(off, CHUNK)], ib[b].at[0], sem_i[b])
        pltpu.async_copy(edges.at[1, pl.ds(off, CHUNK)], ib[b].at[1], sem_i[b])

    def wait_idx(b):
        pltpu.make_async_copy(edges.at[0, pl.ds(0, CHUNK)], ib[b].at[0],
                              sem_i[b]).wait()
        pltpu.make_async_copy(edges.at[1, pl.ds(0, CHUNK)], ib[b].at[1],
                              sem_i[b]).wait()

    def issue_gathers(b):
        pltpu.async_copy(sa_h.at[ib[b].at[0]], sa_v[b], sem_g[b])
        pltpu.async_copy(sb_h.at[ib[b].at[1]], sb_v[b], sem_g[b])
        pltpu.async_copy(tbl_h.at[ib[b].at[1]], rows_v[b], sem_g[b])

    def wait_gathers(b):
        pltpu.make_async_copy(sa_h.at[ib[b].at[0]], sa_v[b], sem_g[b]).wait()
        pltpu.make_async_copy(sb_h.at[ib[b].at[1]], sb_v[b], sem_g[b]).wait()
        pltpu.make_async_copy(tbl_h.at[ib[b].at[1]], rows_v[b], sem_g[b]).wait()

    def issue_scatter(b):
        pltpu.async_copy(scaled_v[b], acc_sh.at[sc_idx[b]], sem_s[b], add=True)
        pltpu.async_copy(w_v[b], rs_sh.at[sc_idx[b]], sem_s[b], add=True)

    def wait_scatter(b):
        pltpu.make_async_copy(scaled_v[b], acc_sh.at[sc_idx[b]], sem_s[b],
                              ).wait()
        pltpu.make_async_copy(w_v[b], rs_sh.at[sc_idx[b]], sem_s[b]).wait()

    issue_idx(0, 0)
    wait_idx(0)
    issue_gathers(0)

    @pl.when(1 < n_my)
    def _p1():
        issue_idx(1, 1)

    def pair_body(g, _):
        for b in (0, 1):
            c = 2 * g + b
            o = 1 - b

            @pl.when(c < n_my)
            def _process():
                wait_gathers(b)

                @pl.when(c >= 2)
                def _drain():
                    wait_scatter(b)

                wregs = []
                for gi in range(CHUNK // L):
                    sc_idx[b][pl.ds(gi * L, L)] = ib[b][0, pl.ds(gi * L, L)]
                    z = sa_v[b][pl.ds(gi * L, L)] + sb_v[b][pl.ds(gi * L, L)]
                    el = jnp.where(z > 0, z, 0.1 * (jnp.exp(z) - 1.0))
                    w = jnp.exp(el)
                    w_v[b][pl.ds(gi * L, L)] = w
                    wregs.append(w)

                @pl.when(c + 2 < n_my)
                def _prefetch_idx():
                    issue_idx(c + 2, b)

                @pl.when(c + 1 < n_my)
                def _prefetch():
                    wait_idx(o)
                    issue_gathers(o)

                for gi in range(CHUNK // L):
                    for u in range(L):
                        e = gi * L + u
                        wv = lax.gather(wregs[gi],
                                        jnp.full((L, 1), u, jnp.int32),
                                        _bcast_dnums, (1,),
                                        mode=lax.GatherScatterMode.PROMISE_IN_BOUNDS)
                        for kk in range(D // L):
                            scaled_v[b][e, pl.ds(kk * L, L)] = (
                                rows_v[b][e, pl.ds(kk * L, L)] * wv)

                issue_scatter(b)
        return 0

    lax.fori_loop(0, (n_my + 1) // 2, pair_body, 0)
    wait_scatter(0)
    wait_scatter(1)
    plsc.subcore_barrier()
    pltpu.sync_copy(acc_sh.at[pl.ds(sid * ROWS_PER_TILE, ROWS_PER_TILE)],
                    out_acc.at[cid, pl.ds(sid * ROWS_PER_TILE, ROWS_PER_TILE)])
    pltpu.sync_copy(rs_sh.at[pl.ds(sid * ROWS_PER_TILE, ROWS_PER_TILE)],
                    out_rs.at[cid, pl.ds(sid * ROWS_PER_TILE, ROWS_PER_TILE)])


_sc_agg = pl.kernel(
    _sc_body,
    out_type=[jax.ShapeDtypeStruct((NC, NPAD, D), jnp.float32),
              jax.ShapeDtypeStruct((NC, NPAD), jnp.float32)],
    mesh=plsc.VectorSubcoreMesh(core_axis_name="c", subcore_axis_name="s",
                                num_cores=NC, num_subcores=NS),
    compiler_params=pltpu.CompilerParams(needs_layout_passes=False),
    scratch_types=[
        [pltpu.VMEM((2, CHUNK), jnp.int32)] * 2,
        [pltpu.VMEM((CHUNK,), jnp.int32)] * 2,
        [pltpu.VMEM((CHUNK,), jnp.float32)] * 2,
        [pltpu.VMEM((CHUNK,), jnp.float32)] * 2,
        [pltpu.VMEM((CHUNK,), jnp.float32)] * 2,
        [pltpu.VMEM((CHUNK, D), jnp.float32)] * 2,
        [pltpu.VMEM((CHUNK, D), jnp.float32)] * 2,
        pltpu.VMEM((L, D), jnp.float32),
        pltpu.VMEM((ROWS_PER_TILE,), jnp.float32),
        [pltpu.SemaphoreType.DMA] * 2,
        [pltpu.SemaphoreType.DMA] * 2,
        [pltpu.SemaphoreType.DMA] * 2,
        pltpu.VMEM_SHARED((NPAD, D), jnp.float32),
        pltpu.VMEM_SHARED((NPAD,), jnp.float32),
    ],
)


def _final_body(feat, accp, accn, rsp_r, rsn_r, w1a, w1b, w1c, bu1, pw, w2,
                bu2, out):
    rsp = rsp_r[0] + rsp_r[1]
    rsp = jnp.where(rsp == 0.0, 1.0, rsp)
    mp = (accp[0] + accp[1]) / rsp
    rsn = rsn_r[0] + rsn_r[1]
    rsn = jnp.where(rsn == 0.0, 1.0, rsn)
    mn = (accn[0] + accn[1]) / rsn
    h = (jnp.dot(feat[...], w1a[...], precision=_HI,
                 preferred_element_type=jnp.float32)
         + jnp.dot(mp, w1b[...], precision=_HI,
                   preferred_element_type=jnp.float32)
         + jnp.dot(mn, w1c[...], precision=_HI,
                   preferred_element_type=jnp.float32)
         + bu1[...])
    h = jnp.where(h > 0, h, pw[0, 0] * h)
    out[...] = jnp.dot(h, w2[...], precision=_HI,
                       preferred_element_type=jnp.float32) + bu2[...]


def _final(feat, accp, accn, rsp, rsn, w1a, w1b, w1c, bu1, pw, w2, bu2):
    full = lambda shape: pl.BlockSpec(shape, lambda i: tuple(0 for _ in shape))
    accblk = pl.BlockSpec((NC, BLK, D), lambda i: (0, i, 0))
    rsblk = pl.BlockSpec((NC, BLK, 1), lambda i: (0, i, 0))
    return pl.pallas_call(
        _final_body,
        grid=(GRID,),
        in_specs=[pl.BlockSpec((BLK, D), lambda i: (i, 0)), accblk, accblk,
                  rsblk, rsblk,
                  full((D, 2 * D)), full((D, 2 * D)), full((D, 2 * D)),
                  full((1, 2 * D)), full((1, 1)), full((2 * D, D)),
                  full((1, D))],
        out_specs=pl.BlockSpec((BLK, D), lambda i: (i, 0)),
        out_shape=jax.ShapeDtypeStruct((N, D), jnp.float32),
    )(feat, accp, accn, rsp, rsn, w1a, w1b, w1c, bu1, pw, w2, bu2)


def kernel(feature_a, feature_b, edge_ab_pos, edge_ab_neg, edge_ba_pos,
           edge_ba_neg, Wm_abp, bm_abp, a_abp, Wm_abn, bm_abn, a_abn,
           Wm_bap, bm_bap, a_bap, Wm_ban, bm_ban, a_ban,
           Wu1, bu1, pw, Wu2, bu2):
    f32 = jnp.float32
    wms = [Wm_abp, Wm_abn, Wm_bap, Wm_ban]
    bms = [b.reshape(1, D).astype(f32) for b in (bm_abp, bm_abn, bm_bap, bm_ban)]
    avs = [a_abp, a_abn, a_bap, a_ban]
    a1s = [a[:D, 0].reshape(1, D).astype(f32) for a in avs]
    a2s = [a[D:, 0].reshape(1, D).astype(f32) for a in avs]

    t0, t1, t2, t3, vecs = _prep(feature_a, feature_b, wms, bms, a1s, a2s)
    tables = (t0, t1, t2, t3)
    edges = (edge_ab_pos.astype(jnp.int32), edge_ab_neg.astype(jnp.int32),
             edge_ba_pos.astype(jnp.int32), edge_ba_neg.astype(jnp.int32))

    accs, rss = [], []
    for i in range(4):
        sa = vecs[2 * i]
        sb = vecs[2 * i + 1]
        acc, rs = _sc_agg(edges[i], sa, sb, tables[i])
        accs.append(acc)
        rss.append(rs.reshape(NC, NPAD, 1))

    w1a = Wu1[:D]
    w1b = Wu1[D:2 * D]
    w1c = Wu1[2 * D:]
    bu1r = bu1.reshape(1, 2 * D)
    bu2r = bu2.reshape(1, D)
    pwr = pw.reshape(1, 1)
    new_a = _final(feature_a, accs[0], accs[1], rss[0], rss[1], w1a, w1b,
                   w1c, bu1r, pwr, Wu2, bu2r)
    new_b = _final(feature_b, accs[2], accs[3], rss[2], rss[3], w1a, w1b,
                   w1c, bu1r, pwr, Wu2, bu2r)
    return (new_a, new_b)
